# baseline (speedup 1.0000x reference)
import functools

import jax
import jax.numpy as jnp
from jax import lax
from jax.experimental import pallas as pl
from jax.experimental.pallas import tpu as pltpu
from jax.experimental.pallas import tpu_sc as plsc

N_NODES = 10000
N_PAD = 10240
N_EDGES = 160000
D = 256
DH = 128
NC = 2
NS = 16
EP = N_EDGES // NS
G = 80
NCH = EP // G
RSH = N_PAD // NS
ZR = 128

_f32 = jnp.float32
_bf16 = jnp.bfloat16
_i32 = jnp.int32


def _norm_from_deg(deg):
    return jnp.where(deg > 0, lax.rsqrt(jnp.maximum(deg, 1.0)), 0.0)


@functools.partial(
    pl.kernel,
    out_type=[
        jax.ShapeDtypeStruct((N_PAD,), _f32),
        jax.ShapeDtypeStruct((N_PAD,), _f32),
    ],
    mesh=plsc.VectorSubcoreMesh(core_axis_name="c", subcore_axis_name="s"),
    scratch_types=[
        pltpu.VMEM((NCH, G), _i32),
        pltpu.VMEM((G,), _f32),
        pltpu.VMEM((RSH,), _f32),
        pltpu.VMEM_SHARED((N_PAD,), _f32),
    ],
)
def _sc_degrees(srcr, dstr, dout, din, idx_v, ones_v, buf_v, acc_sh):
    c = lax.axis_index("c")
    s = lax.axis_index("s")

    @pl.when(c == 0)
    def _():
        pltpu.sync_copy(srcr.at[s], idx_v)

    @pl.when(c == 1)
    def _():
        pltpu.sync_copy(dstr.at[s], idx_v)

    one16 = jnp.ones((16,), _f32)
    zero16 = jnp.zeros((16,), _f32)
    for k in range(G // 16):
        ones_v[pl.ds(k * 16, 16)] = one16

    def _zb(k, carry):
        buf_v[pl.ds(k * 16, 16)] = zero16
        return carry

    lax.fori_loop(0, RSH // 16, _zb, None)
    pltpu.sync_copy(buf_v, acc_sh.at[pl.ds(s * RSH, RSH)])
    plsc.subcore_barrier()

    def _body(i, carry):
        pltpu.sync_copy(ones_v, acc_sh.at[idx_v.at[i]], add=True)
        return carry

    lax.fori_loop(0, NCH, _body, None)
    plsc.subcore_barrier()

    pltpu.sync_copy(acc_sh.at[pl.ds(s * RSH, RSH)], buf_v)

    @pl.when(c == 0)
    def _():
        pltpu.sync_copy(buf_v, dout.at[pl.ds(s * RSH, RSH)])

    @pl.when(c == 1)
    def _():
        pltpu.sync_copy(buf_v, din.at[pl.ds(s * RSH, RSH)])


GC = 128
E_PAD = 163840
EPS = E_PAD // NS
NCHQ = EPS // GC
NB = 4


@functools.partial(
    pl.kernel,
    out_type=[jax.ShapeDtypeStruct((N_PAD, DH), _bf16) for _ in range(2)],
    mesh=plsc.VectorSubcoreMesh(core_axis_name="c", subcore_axis_name="s"),
    scratch_types=[
        pltpu.VMEM((NCHQ, GC), _i32),
        pltpu.VMEM((NCHQ, GC), _i32),
        [pltpu.VMEM((GC, DH), _bf16) for _ in range(NB)],
        pltpu.VMEM((ZR, DH), _bf16),
        pltpu.VMEM((ZR, DH), _bf16),
        [pltpu.SemaphoreType.DMA for _ in range(NB)],
        pltpu.VMEM_SHARED((N_PAD, DH), _bf16),
    ],
    compiler_params=pltpu.CompilerParams(use_tc_tiling_on_sc=False),
)
def _sc_scatter(
    t0, t1, zeros_h, srcr, dstr, o0, o1,
    src_v, dst_v, rbs, zbuf, cbuf, sems, acc_sh,
):
    c = lax.axis_index("c")
    s = lax.axis_index("s")

    pltpu.sync_copy(srcr.at[s], src_v)
    pltpu.sync_copy(dstr.at[s], dst_v)
    pltpu.sync_copy(zeros_h, zbuf)

    def _gather(i, b):
        @pl.when(c == 0)
        def _():
            pltpu.async_copy(t0.at[src_v.at[i]], rbs[b], sems[b])

        @pl.when(c == 1)
        def _():
            pltpu.async_copy(t1.at[src_v.at[i]], rbs[b], sems[b])

    for b in range(NB):
        _gather(b, b)
    for r in range(RSH // ZR):
        pltpu.sync_copy(zbuf, acc_sh.at[pl.ds(s * RSH + r * ZR, ZR)])
    plsc.subcore_barrier()

    def _body(j, carry):
        for b in range(NB):
            i = NB * j + b
            pltpu.make_async_copy(t0.at[src_v.at[i]], rbs[b], sems[b]).wait()
            pltpu.sync_copy(rbs[b], acc_sh.at[dst_v.at[i]], add=True)

            @pl.when(i + NB < NCHQ)
            def _():
                _gather(i + NB, b)

        return carry

    lax.fori_loop(0, NCHQ // NB, _body, None)
    plsc.subcore_barrier()

    for r in range(RSH // ZR):
        sl = pl.ds(s * RSH + r * ZR, ZR)
        pltpu.sync_copy(acc_sh.at[sl], cbuf)

        @pl.when(c == 0)
        def _():
            pltpu.sync_copy(cbuf, o0.at[sl])

        @pl.when(c == 1)
        def _():
            pltpu.sync_copy(cbuf, o1.at[sl])


_RB = 1000


def _tc_first_body(x_ref, w_ref, dout_ref, out_ref):
    no = _norm_from_deg(dout_ref[...])
    res = jnp.dot(x_ref[...], w_ref[...], preferred_element_type=_f32) * no
    for h in range(2):
        out_ref[h] = res[:, h * DH : (h + 1) * DH].astype(_bf16)


def _tc_mid_body(a0_ref, a1_ref, din_ref, b_ref, w_ref, dout_ref, out_ref):
    ni = _norm_from_deg(din_ref[...])
    no = _norm_from_deg(dout_ref[...])
    a = (
        jnp.concatenate([a0_ref[...], a1_ref[...]], axis=1).astype(_f32) * ni
        + b_ref[...]
    )
    h_act = jnp.maximum(a, 0.0)
    res = jnp.dot(h_act, w_ref[...], preferred_element_type=_f32) * no
    for h in range(2):
        out_ref[h] = res[:, h * DH : (h + 1) * DH].astype(_bf16)


def _tc_last_body(a0_ref, a1_ref, din_ref, b_ref, out_ref):
    ni = _norm_from_deg(din_ref[...])
    out_ref[...] = (
        jnp.concatenate([a0_ref[...], a1_ref[...]], axis=1).astype(_f32) * ni
        + b_ref[...]
    )


def _half_spec():
    return pl.BlockSpec((_RB, DH), lambda i: (i, 0))


def _tc_first(x, w, dout):
    return pl.pallas_call(
        _tc_first_body,
        grid=(N_NODES // _RB,),
        in_specs=[
            pl.BlockSpec((_RB, D), lambda i: (i, 0)),
            pl.BlockSpec((D, D), lambda i: (0, 0)),
            pl.BlockSpec((_RB, 1), lambda i: (i, 0)),
        ],
        out_specs=pl.BlockSpec((2, _RB, DH), lambda i: (0, i, 0)),
        out_shape=jax.ShapeDtypeStruct((2, N_NODES, DH), _bf16),
    )(x, w, dout)


def _tc_mid(a0, a1, din, b, w, dout):
    return pl.pallas_call(
        _tc_mid_body,
        grid=(N_NODES // _RB,),
        in_specs=[
            _half_spec(),
            _half_spec(),
            pl.BlockSpec((_RB, 1), lambda i: (i, 0)),
            pl.BlockSpec((1, D), lambda i: (0, 0)),
            pl.BlockSpec((D, D), lambda i: (0, 0)),
            pl.BlockSpec((_RB, 1), lambda i: (i, 0)),
        ],
        out_specs=pl.BlockSpec((2, _RB, DH), lambda i: (0, i, 0)),
        out_shape=jax.ShapeDtypeStruct((2, N_NODES, DH), _bf16),
    )(a0, a1, din, b, w, dout)


def _tc_last(a0, a1, din, b):
    return pl.pallas_call(
        _tc_last_body,
        grid=(N_NODES // _RB,),
        in_specs=[
            _half_spec(),
            _half_spec(),
            pl.BlockSpec((_RB, 1), lambda i: (i, 0)),
            pl.BlockSpec((1, D), lambda i: (0, 0)),
        ],
        out_specs=pl.BlockSpec((_RB, D), lambda i: (i, 0)),
        out_shape=jax.ShapeDtypeStruct((N_NODES, D), _f32),
    )(a0, a1, din, b)


def kernel(x, edge_index, W0, b0, W1, b1, W2, b2):
    src = edge_index[0].astype(_i32)
    dst = edge_index[1].astype(_i32)
    srcr = src.reshape(NS, NCH, G)
    dstr = dst.reshape(NS, NCH, G)

    npad = E_PAD - N_EDGES
    pad_iota = jnp.arange(npad, dtype=_i32)
    src_p = jnp.concatenate([src, pad_iota % N_NODES]).reshape(NS, NCHQ, GC)
    dst_p = jnp.concatenate([dst, N_NODES + pad_iota % (N_PAD - N_NODES)]).reshape(
        NS, NCHQ, GC
    )

    zeros_h = jnp.zeros((ZR, DH), _bf16)

    dout_p, din_p = _sc_degrees(srcr, dstr)
    dout = dout_p[:N_NODES].reshape(N_NODES, 1)
    din = din_p[:N_NODES].reshape(N_NODES, 1)

    b0r = b0.reshape(1, D)
    b1r = b1.reshape(1, D)
    b2r = b2.reshape(1, D)

    p = _tc_first(x, W0, dout)
    a = _sc_scatter(p[0], p[1], zeros_h, src_p, dst_p)
    a = [t[:N_NODES] for t in a]
    p = _tc_mid(a[0], a[1], din, b0r, W1, dout)
    a = _sc_scatter(p[0], p[1], zeros_h, src_p, dst_p)
    a = [t[:N_NODES] for t in a]
    p = _tc_mid(a[0], a[1], din, b1r, W2, dout)
    a = _sc_scatter(p[0], p[1], zeros_h, src_p, dst_p)
    a = [t[:N_NODES] for t in a]
    return _tc_last(a[0], a[1], din, b2r)


# speedup vs baseline: 1.0242x; 1.0242x over previous
import functools

import jax
import jax.numpy as jnp
from jax import lax
from jax.experimental import pallas as pl
from jax.experimental.pallas import tpu as pltpu
from jax.experimental.pallas import tpu_sc as plsc

N_NODES = 10000
N_PAD = 10240
N_EDGES = 160000
D = 256
DH = 128
NC = 2
NS = 16
EP = N_EDGES // NS
G = 80
NCH = EP // G
RSH = N_PAD // NS
ZR = 128

_f32 = jnp.float32
_bf16 = jnp.bfloat16
_i32 = jnp.int32


def _norm_from_deg(deg):
    return jnp.where(deg > 0, lax.rsqrt(jnp.maximum(deg, 1.0)), 0.0)


@functools.partial(
    pl.kernel,
    out_type=[
        jax.ShapeDtypeStruct((N_PAD,), _f32),
        jax.ShapeDtypeStruct((N_PAD,), _f32),
    ],
    mesh=plsc.VectorSubcoreMesh(core_axis_name="c", subcore_axis_name="s"),
    scratch_types=[
        pltpu.VMEM((NCH, G), _i32),
        pltpu.VMEM((G,), _f32),
        pltpu.VMEM((RSH,), _f32),
        pltpu.VMEM_SHARED((N_PAD,), _f32),
    ],
)
def _sc_degrees(srcr, dstr, dout, din, idx_v, ones_v, buf_v, acc_sh):
    c = lax.axis_index("c")
    s = lax.axis_index("s")

    @pl.when(c == 0)
    def _():
        pltpu.sync_copy(srcr.at[s], idx_v)

    @pl.when(c == 1)
    def _():
        pltpu.sync_copy(dstr.at[s], idx_v)

    one16 = jnp.ones((16,), _f32)
    zero16 = jnp.zeros((16,), _f32)
    for k in range(G // 16):
        ones_v[pl.ds(k * 16, 16)] = one16

    def _zb(k, carry):
        buf_v[pl.ds(k * 16, 16)] = zero16
        return carry

    lax.fori_loop(0, RSH // 16, _zb, None)
    pltpu.sync_copy(buf_v, acc_sh.at[pl.ds(s * RSH, RSH)])
    plsc.subcore_barrier()

    def _body(i, carry):
        pltpu.sync_copy(ones_v, acc_sh.at[idx_v.at[i]], add=True)
        return carry

    lax.fori_loop(0, NCH, _body, None)
    plsc.subcore_barrier()

    pltpu.sync_copy(acc_sh.at[pl.ds(s * RSH, RSH)], buf_v)

    @pl.when(c == 0)
    def _():
        pltpu.sync_copy(buf_v, dout.at[pl.ds(s * RSH, RSH)])

    @pl.when(c == 1)
    def _():
        pltpu.sync_copy(buf_v, din.at[pl.ds(s * RSH, RSH)])


GC = 128
E_PAD = 163840
EPS = E_PAD // NS
NCHQ = EPS // GC
NB = 4


@functools.partial(
    pl.kernel,
    out_type=[jax.ShapeDtypeStruct((N_PAD, DH), _bf16) for _ in range(2)],
    mesh=plsc.VectorSubcoreMesh(core_axis_name="c", subcore_axis_name="s"),
    scratch_types=[
        pltpu.VMEM((NCHQ, GC), _i32),
        pltpu.VMEM((NCHQ, GC), _i32),
        [pltpu.VMEM((GC, DH), _bf16) for _ in range(NB)],
        pltpu.VMEM((ZR, DH), _bf16),
        pltpu.VMEM((ZR, DH), _bf16),
        [pltpu.SemaphoreType.DMA for _ in range(NB)],
        pltpu.VMEM_SHARED((N_PAD, DH), _bf16),
    ],
    compiler_params=pltpu.CompilerParams(use_tc_tiling_on_sc=False),
)
def _sc_scatter(
    t0, t1, zeros_h, srcr, dstr, o0, o1,
    src_v, dst_v, rbs, zbuf, cbuf, sems, acc_sh,
):
    c = lax.axis_index("c")
    s = lax.axis_index("s")

    pltpu.sync_copy(srcr.at[s], src_v)
    pltpu.sync_copy(dstr.at[s], dst_v)
    pltpu.sync_copy(zeros_h, zbuf)

    def _gather(i, b):
        @pl.when(c == 0)
        def _():
            pltpu.async_copy(t0.at[src_v.at[i]], rbs[b], sems[b])

        @pl.when(c == 1)
        def _():
            pltpu.async_copy(t1.at[src_v.at[i]], rbs[b], sems[b])

    for b in range(NB):
        _gather(b, b)
    for r in range(RSH // ZR):
        pltpu.sync_copy(zbuf, acc_sh.at[pl.ds(s * RSH + r * ZR, ZR)])
    plsc.subcore_barrier()

    def _body(j, carry):
        for b in range(NB):
            i = NB * j + b
            pltpu.make_async_copy(t0.at[src_v.at[i]], rbs[b], sems[b]).wait()
            pltpu.sync_copy(rbs[b], acc_sh.at[dst_v.at[i]], add=True)

            @pl.when(i + NB < NCHQ)
            def _():
                _gather(i + NB, b)

        return carry

    lax.fori_loop(0, NCHQ // NB, _body, None)
    plsc.subcore_barrier()

    for r in range(RSH // ZR):
        sl = pl.ds(s * RSH + r * ZR, ZR)
        pltpu.sync_copy(acc_sh.at[sl], cbuf)

        @pl.when(c == 0)
        def _():
            pltpu.sync_copy(cbuf, o0.at[sl])

        @pl.when(c == 1)
        def _():
            pltpu.sync_copy(cbuf, o1.at[sl])


_RB = 1000


def _tc_first_body(x_ref, w_ref, dout_ref, out0_ref, out1_ref):
    no = _norm_from_deg(dout_ref[...])
    res = jnp.dot(x_ref[...], w_ref[...], preferred_element_type=_f32) * no
    out0_ref[...] = res[:, :DH].astype(_bf16)
    out1_ref[...] = res[:, DH:].astype(_bf16)


def _tc_mid_body(a0_ref, a1_ref, din_ref, b_ref, w_ref, dout_ref, out0_ref, out1_ref):
    ni = _norm_from_deg(din_ref[...])
    no = _norm_from_deg(dout_ref[...])
    a = (
        jnp.concatenate([a0_ref[...], a1_ref[...]], axis=1).astype(_f32) * ni
        + b_ref[...]
    )
    h_act = jnp.maximum(a, 0.0)
    res = jnp.dot(h_act, w_ref[...], preferred_element_type=_f32) * no
    out0_ref[...] = res[:, :DH].astype(_bf16)
    out1_ref[...] = res[:, DH:].astype(_bf16)


def _tc_last_body(a0_ref, a1_ref, din_ref, b_ref, out_ref):
    ni = _norm_from_deg(din_ref[...])
    out_ref[...] = (
        jnp.concatenate([a0_ref[...], a1_ref[...]], axis=1).astype(_f32) * ni
        + b_ref[...]
    )


def _half_spec():
    return pl.BlockSpec((_RB, DH), lambda i: (i, 0))


def _half_out():
    return [
        pl.BlockSpec((_RB, DH), lambda i: (i, 0)),
        pl.BlockSpec((_RB, DH), lambda i: (i, 0)),
    ]


def _half_out_shape():
    return [
        jax.ShapeDtypeStruct((N_PAD, DH), _bf16),
        jax.ShapeDtypeStruct((N_PAD, DH), _bf16),
    ]


def _tc_first(x, w, dout):
    return pl.pallas_call(
        _tc_first_body,
        grid=(N_NODES // _RB,),
        in_specs=[
            pl.BlockSpec((_RB, D), lambda i: (i, 0)),
            pl.BlockSpec((D, D), lambda i: (0, 0)),
            pl.BlockSpec((_RB, 1), lambda i: (i, 0)),
        ],
        out_specs=_half_out(),
        out_shape=_half_out_shape(),
    )(x, w, dout)


def _tc_mid(a0, a1, din, b, w, dout):
    return pl.pallas_call(
        _tc_mid_body,
        grid=(N_NODES // _RB,),
        in_specs=[
            _half_spec(),
            _half_spec(),
            pl.BlockSpec((_RB, 1), lambda i: (i, 0)),
            pl.BlockSpec((1, D), lambda i: (0, 0)),
            pl.BlockSpec((D, D), lambda i: (0, 0)),
            pl.BlockSpec((_RB, 1), lambda i: (i, 0)),
        ],
        out_specs=_half_out(),
        out_shape=_half_out_shape(),
    )(a0, a1, din, b, w, dout)


def _tc_last(a0, a1, din, b):
    return pl.pallas_call(
        _tc_last_body,
        grid=(N_NODES // _RB,),
        in_specs=[
            _half_spec(),
            _half_spec(),
            pl.BlockSpec((_RB, 1), lambda i: (i, 0)),
            pl.BlockSpec((1, D), lambda i: (0, 0)),
        ],
        out_specs=pl.BlockSpec((_RB, D), lambda i: (i, 0)),
        out_shape=jax.ShapeDtypeStruct((N_NODES, D), _f32),
    )(a0, a1, din, b)


def kernel(x, edge_index, W0, b0, W1, b1, W2, b2):
    src = edge_index[0].astype(_i32)
    dst = edge_index[1].astype(_i32)
    srcr = src.reshape(NS, NCH, G)
    dstr = dst.reshape(NS, NCH, G)

    npad = E_PAD - N_EDGES
    pad_iota = jnp.arange(npad, dtype=_i32)
    src_p = jnp.concatenate([src, pad_iota % N_NODES]).reshape(NS, NCHQ, GC)
    dst_p = jnp.concatenate([dst, N_NODES + pad_iota % (N_PAD - N_NODES)]).reshape(
        NS, NCHQ, GC
    )

    zeros_h = jnp.zeros((ZR, DH), _bf16)

    dout_p, din_p = _sc_degrees(srcr, dstr)
    dout = dout_p.reshape(N_PAD, 1)
    din = din_p.reshape(N_PAD, 1)

    b0r = b0.reshape(1, D)
    b1r = b1.reshape(1, D)
    b2r = b2.reshape(1, D)

    p0, p1 = _tc_first(x, W0, dout)
    a0, a1 = _sc_scatter(p0, p1, zeros_h, src_p, dst_p)
    p0, p1 = _tc_mid(a0, a1, din, b0r, W1, dout)
    a0, a1 = _sc_scatter(p0, p1, zeros_h, src_p, dst_p)
    p0, p1 = _tc_mid(a0, a1, din, b1r, W2, dout)
    a0, a1 = _sc_scatter(p0, p1, zeros_h, src_p, dst_p)
    return _tc_last(a0, a1, din, b2r)


# speedup vs baseline: 1.0248x; 1.0006x over previous
import functools

import jax
import jax.numpy as jnp
from jax import lax
from jax.experimental import pallas as pl
from jax.experimental.pallas import tpu as pltpu
from jax.experimental.pallas import tpu_sc as plsc

N_NODES = 10000
N_PAD = 10240
N_EDGES = 160000
D = 256
DH = 128
NC = 2
NS = 16
EP = N_EDGES // NS
G = 80
NCH = EP // G
RSH = N_PAD // NS
ZR = 128

_f32 = jnp.float32
_bf16 = jnp.bfloat16
_i32 = jnp.int32


def _norm_from_deg(deg):
    return jnp.where(deg > 0, lax.rsqrt(jnp.maximum(deg, 1.0)), 0.0)


@functools.partial(
    pl.kernel,
    out_type=[
        jax.ShapeDtypeStruct((N_PAD,), _f32),
        jax.ShapeDtypeStruct((N_PAD,), _f32),
    ],
    mesh=plsc.VectorSubcoreMesh(core_axis_name="c", subcore_axis_name="s"),
    scratch_types=[
        pltpu.VMEM((NCH, G), _i32),
        pltpu.VMEM((G,), _f32),
        pltpu.VMEM((RSH,), _f32),
        pltpu.VMEM_SHARED((N_PAD,), _f32),
    ],
)
def _sc_degrees(srcr, dstr, dout, din, idx_v, ones_v, buf_v, acc_sh):
    c = lax.axis_index("c")
    s = lax.axis_index("s")

    @pl.when(c == 0)
    def _():
        pltpu.sync_copy(srcr.at[s], idx_v)

    @pl.when(c == 1)
    def _():
        pltpu.sync_copy(dstr.at[s], idx_v)

    one16 = jnp.ones((16,), _f32)
    zero16 = jnp.zeros((16,), _f32)
    for k in range(G // 16):
        ones_v[pl.ds(k * 16, 16)] = one16

    def _zb(k, carry):
        buf_v[pl.ds(k * 16, 16)] = zero16
        return carry

    lax.fori_loop(0, RSH // 16, _zb, None)
    pltpu.sync_copy(buf_v, acc_sh.at[pl.ds(s * RSH, RSH)])
    plsc.subcore_barrier()

    def _body(i, carry):
        pltpu.sync_copy(ones_v, acc_sh.at[idx_v.at[i]], add=True)
        return carry

    lax.fori_loop(0, NCH, _body, None)
    plsc.subcore_barrier()

    pltpu.sync_copy(acc_sh.at[pl.ds(s * RSH, RSH)], buf_v)

    @pl.when(c == 0)
    def _():
        pltpu.sync_copy(buf_v, dout.at[pl.ds(s * RSH, RSH)])

    @pl.when(c == 1)
    def _():
        pltpu.sync_copy(buf_v, din.at[pl.ds(s * RSH, RSH)])


GC = 128
E_PAD = 163840
EPS = E_PAD // NS
NCHQ = EPS // GC
NB = 4


@functools.partial(
    pl.kernel,
    out_type=[jax.ShapeDtypeStruct((N_PAD, DH), _bf16) for _ in range(2)],
    mesh=plsc.VectorSubcoreMesh(core_axis_name="c", subcore_axis_name="s"),
    scratch_types=[
        pltpu.VMEM((NCHQ, GC), _i32),
        pltpu.VMEM((NCHQ, GC), _i32),
        [pltpu.VMEM((GC, DH), _bf16) for _ in range(NB)],
        pltpu.VMEM((ZR, DH), _bf16),
        pltpu.VMEM((ZR, DH), _bf16),
        [pltpu.SemaphoreType.DMA for _ in range(NB)],
        pltpu.VMEM_SHARED((N_PAD, DH), _bf16),
    ],
    compiler_params=pltpu.CompilerParams(use_tc_tiling_on_sc=False),
)
def _sc_scatter(
    t0, t1, zeros_h, srcr, dstr, o0, o1,
    src_v, dst_v, rbs, zbuf, cbuf, sems, acc_sh,
):
    c = lax.axis_index("c")
    s = lax.axis_index("s")

    pltpu.sync_copy(srcr.at[s], src_v)
    pltpu.sync_copy(dstr.at[s], dst_v)
    pltpu.sync_copy(zeros_h, zbuf)

    def _gather(i, b):
        @pl.when(c == 0)
        def _():
            pltpu.async_copy(t0.at[src_v.at[i]], rbs[b], sems[b])

        @pl.when(c == 1)
        def _():
            pltpu.async_copy(t1.at[src_v.at[i]], rbs[b], sems[b])

    for b in range(NB):
        _gather(b, b)
    for r in range(RSH // ZR):
        pltpu.sync_copy(zbuf, acc_sh.at[pl.ds(s * RSH + r * ZR, ZR)])
    plsc.subcore_barrier()

    def _body(j, carry):
        for b in range(NB):
            i = NB * j + b
            pltpu.make_async_copy(t0.at[src_v.at[i]], rbs[b], sems[b]).wait()
            pltpu.sync_copy(rbs[b], acc_sh.at[dst_v.at[i]], add=True)

            @pl.when(i + NB < NCHQ)
            def _():
                _gather(i + NB, b)

        return carry

    lax.fori_loop(0, NCHQ // NB, _body, None)
    plsc.subcore_barrier()

    for r in range(RSH // ZR):
        sl = pl.ds(s * RSH + r * ZR, ZR)
        pltpu.sync_copy(acc_sh.at[sl], cbuf)

        @pl.when(c == 0)
        def _():
            pltpu.sync_copy(cbuf, o0.at[sl])

        @pl.when(c == 1)
        def _():
            pltpu.sync_copy(cbuf, o1.at[sl])


_RB = 1000


def _tc_first_body(x_ref, w_ref, dout_ref, out0_ref, out1_ref):
    no = _norm_from_deg(dout_ref[...])
    res = (
        jnp.dot(
            x_ref[...].astype(_bf16),
            w_ref[...].astype(_bf16),
            preferred_element_type=_f32,
        )
        * no
    )
    out0_ref[...] = res[:, :DH].astype(_bf16)
    out1_ref[...] = res[:, DH:].astype(_bf16)


def _tc_mid_body(a0_ref, a1_ref, din_ref, b_ref, w_ref, dout_ref, out0_ref, out1_ref):
    ni = _norm_from_deg(din_ref[...])
    no = _norm_from_deg(dout_ref[...])
    a = (
        jnp.concatenate([a0_ref[...], a1_ref[...]], axis=1).astype(_f32) * ni
        + b_ref[...]
    )
    h_act = jnp.maximum(a, 0.0)
    res = (
        jnp.dot(
            h_act.astype(_bf16),
            w_ref[...].astype(_bf16),
            preferred_element_type=_f32,
        )
        * no
    )
    out0_ref[...] = res[:, :DH].astype(_bf16)
    out1_ref[...] = res[:, DH:].astype(_bf16)


def _tc_last_body(a0_ref, a1_ref, din_ref, b_ref, out_ref):
    ni = _norm_from_deg(din_ref[...])
    out_ref[...] = (
        jnp.concatenate([a0_ref[...], a1_ref[...]], axis=1).astype(_f32) * ni
        + b_ref[...]
    )


def _half_spec():
    return pl.BlockSpec((_RB, DH), lambda i: (i, 0))


def _half_out():
    return [
        pl.BlockSpec((_RB, DH), lambda i: (i, 0)),
        pl.BlockSpec((_RB, DH), lambda i: (i, 0)),
    ]


def _half_out_shape():
    return [
        jax.ShapeDtypeStruct((N_PAD, DH), _bf16),
        jax.ShapeDtypeStruct((N_PAD, DH), _bf16),
    ]


def _tc_first(x, w, dout):
    return pl.pallas_call(
        _tc_first_body,
        grid=(N_NODES // _RB,),
        in_specs=[
            pl.BlockSpec((_RB, D), lambda i: (i, 0)),
            pl.BlockSpec((D, D), lambda i: (0, 0)),
            pl.BlockSpec((_RB, 1), lambda i: (i, 0)),
        ],
        out_specs=_half_out(),
        out_shape=_half_out_shape(),
    )(x, w, dout)


def _tc_mid(a0, a1, din, b, w, dout):
    return pl.pallas_call(
        _tc_mid_body,
        grid=(N_NODES // _RB,),
        in_specs=[
            _half_spec(),
            _half_spec(),
            pl.BlockSpec((_RB, 1), lambda i: (i, 0)),
            pl.BlockSpec((1, D), lambda i: (0, 0)),
            pl.BlockSpec((D, D), lambda i: (0, 0)),
            pl.BlockSpec((_RB, 1), lambda i: (i, 0)),
        ],
        out_specs=_half_out(),
        out_shape=_half_out_shape(),
    )(a0, a1, din, b, w, dout)


def _tc_last(a0, a1, din, b):
    return pl.pallas_call(
        _tc_last_body,
        grid=(N_NODES // _RB,),
        in_specs=[
            _half_spec(),
            _half_spec(),
            pl.BlockSpec((_RB, 1), lambda i: (i, 0)),
            pl.BlockSpec((1, D), lambda i: (0, 0)),
        ],
        out_specs=pl.BlockSpec((_RB, D), lambda i: (i, 0)),
        out_shape=jax.ShapeDtypeStruct((N_NODES, D), _f32),
    )(a0, a1, din, b)


def kernel(x, edge_index, W0, b0, W1, b1, W2, b2):
    src = edge_index[0].astype(_i32)
    dst = edge_index[1].astype(_i32)
    srcr = src.reshape(NS, NCH, G)
    dstr = dst.reshape(NS, NCH, G)

    npad = E_PAD - N_EDGES
    pad_iota = jnp.arange(npad, dtype=_i32)
    src_p = jnp.concatenate([src, pad_iota % N_NODES]).reshape(NS, NCHQ, GC)
    dst_p = jnp.concatenate([dst, N_NODES + pad_iota % (N_PAD - N_NODES)]).reshape(
        NS, NCHQ, GC
    )

    zeros_h = jnp.zeros((ZR, DH), _bf16)

    dout_p, din_p = _sc_degrees(srcr, dstr)
    dout = dout_p.reshape(N_PAD, 1)
    din = din_p.reshape(N_PAD, 1)

    b0r = b0.reshape(1, D)
    b1r = b1.reshape(1, D)
    b2r = b2.reshape(1, D)

    p0, p1 = _tc_first(x, W0, dout)
    a0, a1 = _sc_scatter(p0, p1, zeros_h, src_p, dst_p)
    p0, p1 = _tc_mid(a0, a1, din, b0r, W1, dout)
    a0, a1 = _sc_scatter(p0, p1, zeros_h, src_p, dst_p)
    p0, p1 = _tc_mid(a0, a1, din, b1r, W2, dout)
    a0, a1 = _sc_scatter(p0, p1, zeros_h, src_p, dst_p)
    return _tc_last(a0, a1, din, b2r)
